# baseline (speedup 1.0000x reference)
import functools

import jax
import jax.numpy as jnp
import numpy as np
from jax import lax
from jax.experimental import pallas as pl
from jax.experimental.pallas import tpu as pltpu
from jax.experimental.pallas import tpu_sc as plsc

N_B, C_IN, T_LEN = 8, 20, 16384
C_FEAT, KERNEL, STRIDE = 256, 64, 64
L = T_LEN // STRIDE
N_HEADS, DH = 8, 32
TEMP, NUM_NEG, EPS = 0.5, 100, 1e-8
P_TOK = 384
LW = 264
JPAD = 112
ROWS = N_B * L
N_TILES = 32
RPT = ROWS // N_TILES


def _eval_mask_np():
    mask = np.zeros((N_B, L), dtype=bool)
    half = max(1, int(L * 0.1 * 0.5))
    seeds = (L // half) * np.arange(half, dtype=np.int64)
    inds = set()
    for s in seeds:
        for i in range(int(s), int(s) + 6):
            if i < L:
                inds.add(i)
    mask[:, sorted(inds)] = True
    return mask


_MASK_NP = _eval_mask_np()


def _tc_body(x_ref, wc_ref, bc_ref, mt_ref, st_ref, mcol_ref, wqkv_ref,
             wo_ref, zT_ref, coT_ref, m_ref):
    f32 = jnp.float32
    xr = x_ref[0]
    zt = jnp.dot(xr, wc_ref[...], preferred_element_type=f32)
    zt = jax.nn.gelu(zt + bc_ref[...])
    zT_ref[0] = zt.T

    mcol = mcol_ref[...]
    tokmid = zt * (1.0 - mcol) + mt_ref[...] * mcol
    tok = jnp.concatenate(
        [st_ref[...], tokmid, jnp.zeros((P_TOK - 1 - L, C_FEAT), f32)],
        axis=0)
    qkv = jnp.dot(tok, wqkv_ref[...], preferred_element_type=f32)
    q = qkv[:, 0:C_FEAT]
    k = qkv[:, C_FEAT:2 * C_FEAT]
    v = qkv[:, 2 * C_FEAT:3 * C_FEAT]

    scale = f32(1.0 / np.sqrt(DH))
    o_parts = []
    for h in range(N_HEADS):
        sl = slice(h * DH, (h + 1) * DH)
        qh, kh, vh = q[:, sl], k[:, sl], v[:, sl]
        sc = lax.dot_general(qh, kh, (((1,), (1,)), ((), ())),
                             preferred_element_type=f32) * scale
        e = jnp.exp(sc)
        s = jnp.sum(e, axis=1, keepdims=True) - f32(P_TOK - 1 - L)
        ev = jnp.dot(e, vh, preferred_element_type=f32)
        o_parts.append(ev / s)
    o = jnp.concatenate(o_parts, axis=1)

    wo = wo_ref[...]
    co = jnp.dot(o, wo, preferred_element_type=f32)
    coT = lax.dot_general(wo, o, (((0,), (1,)), ((), ())),
                          preferred_element_type=f32)
    coT_ref[0] = coT[:, :L + 1]

    cq = co[1:L + 1, :]
    S = lax.dot_general(cq, zt, (((1,), (1,)), ((), ())),
                        preferred_element_type=f32)
    ones = jnp.ones((1, L), f32)
    zn2 = lax.dot_general(ones, zt * zt, (((1,), (1,)), ((), ())),
                          preferred_element_type=f32)
    cq2 = jnp.sum(cq * cq, axis=1, keepdims=True)
    den = jnp.maximum(jnp.sqrt(cq2) * jnp.sqrt(zn2), f32(EPS))
    Sp = S / den * f32(1.0 / TEMP)
    T = cq2 + zn2 - 2.0 * S
    eqm = T < (cq2 + zn2) * f32(1e-6)
    Mneg = jnp.where(eqm, f32(-jnp.inf), Sp)
    eye = (lax.broadcasted_iota(jnp.int32, (L, L), 0)
           == lax.broadcasted_iota(jnp.int32, (L, L), 1))
    diag = jnp.sum(jnp.where(eye, Sp, f32(0.0)), axis=1, keepdims=True)
    m_ref[0] = jnp.concatenate(
        [Mneg, diag, jnp.zeros((L, LW - L - 1), f32)], axis=1)


def _tc(xr, wc, bc, mt, st, mcol, wqkv, wo):
    return pl.pallas_call(
        _tc_body,
        grid=(N_B,),
        in_specs=[
            pl.BlockSpec((1, L, C_IN * KERNEL), lambda i: (i, 0, 0)),
            pl.BlockSpec((C_IN * KERNEL, C_FEAT), lambda i: (0, 0)),
            pl.BlockSpec((1, C_FEAT), lambda i: (0, 0)),
            pl.BlockSpec((1, C_FEAT), lambda i: (0, 0)),
            pl.BlockSpec((1, C_FEAT), lambda i: (0, 0)),
            pl.BlockSpec((L, 1), lambda i: (0, 0)),
            pl.BlockSpec((C_FEAT, 3 * C_FEAT), lambda i: (0, 0)),
            pl.BlockSpec((C_FEAT, C_FEAT), lambda i: (0, 0)),
        ],
        out_specs=[
            pl.BlockSpec((1, C_FEAT, L), lambda i: (i, 0, 0)),
            pl.BlockSpec((1, C_FEAT, L + 1), lambda i: (i, 0, 0)),
            pl.BlockSpec((1, L, LW), lambda i: (i, 0, 0)),
        ],
        out_shape=[
            jax.ShapeDtypeStruct((N_B, C_FEAT, L), jnp.float32),
            jax.ShapeDtypeStruct((N_B, C_FEAT, L + 1), jnp.float32),
            jax.ShapeDtypeStruct((N_B, L, LW), jnp.float32),
        ],
        compiler_params=pltpu.CompilerParams(
            dimension_semantics=("arbitrary",)),
    )(xr, wc, bc, mt, st, mcol, wqkv, wo)


_WORDS_M = RPT * LW
_WORDS_J = RPT * JPAD


@functools.cache
def _sc_gather_kernel():
    mesh = plsc.VectorSubcoreMesh(core_axis_name="c", subcore_axis_name="s")

    @functools.partial(
        pl.kernel,
        mesh=mesh,
        out_type=jax.ShapeDtypeStruct((ROWS * JPAD,), jnp.float32),
        scratch_types=[
            pltpu.VMEM((_WORDS_M,), jnp.float32),
            pltpu.VMEM((_WORDS_J,), jnp.int32),
            pltpu.VMEM((_WORDS_J,), jnp.float32),
        ],
        compiler_params=pltpu.CompilerParams(needs_layout_passes=False),
    )
    def sc_gather(m_hbm, neg_hbm, out_hbm, m_v, neg_v, out_v):
        nc = 2
        wid = lax.axis_index("s") * nc + lax.axis_index("c")
        pltpu.sync_copy(m_hbm.at[pl.ds(wid * _WORDS_M, _WORDS_M)], m_v)
        pltpu.sync_copy(neg_hbm.at[pl.ds(wid * _WORDS_J, _WORDS_J)], neg_v)

        def body(r, carry):
            rb = r * JPAD
            off = r * LW
            for j in range(JPAD // 16):
                idx = neg_v[pl.ds(rb + 16 * j, 16)] + off
                out_v[pl.ds(rb + 16 * j, 16)] = plsc.load_gather(m_v, [idx])
            return carry

        lax.fori_loop(0, RPT, body, 0)
        pltpu.sync_copy(out_v, out_hbm.at[pl.ds(wid * _WORDS_J, _WORDS_J)])

    return sc_gather


def _rotl32(x, r):
    return ((x << np.uint32(r)) | (x >> np.uint32(32 - r))).astype(np.uint32)


def _threefry2x32_raw(k1, k2, x1, x2):
    x = [x1.astype(np.uint32).copy(), x2.astype(np.uint32).copy()]
    k1, k2 = np.uint32(k1), np.uint32(k2)
    ks = [k1, k2, np.uint32(k1 ^ k2 ^ np.uint32(0x1BD11BDA))]
    rotations = [(13, 15, 26, 6), (17, 29, 16, 24)]
    with np.errstate(over="ignore"):
        x[0] = (x[0] + ks[0]).astype(np.uint32)
        x[1] = (x[1] + ks[1]).astype(np.uint32)
        for i in range(5):
            for r in rotations[i % 2]:
                x[0] = (x[0] + x[1]).astype(np.uint32)
                x[1] = _rotl32(x[1], r)
                x[1] = (x[1] ^ x[0]).astype(np.uint32)
            x[0] = (x[0] + ks[(i + 1) % 3]).astype(np.uint32)
            x[1] = (x[1] + ks[(i + 2) % 3] + np.uint32(i + 1)).astype(np.uint32)
    return x[0], x[1]


def _threefry2x32_flat(k1, k2, count):
    odd = count.size % 2
    flat = count.ravel().astype(np.uint32)
    if odd:
        flat = np.concatenate([flat, np.zeros(1, np.uint32)])
    h = flat.size // 2
    b1, b2 = _threefry2x32_raw(k1, k2, flat[:h], flat[h:])
    out = np.concatenate([b1, b2])
    if odd:
        out = out[:-1]
    return out


def _np_randint_0_255(shape, seed=1):
    import jax._src.config as _jc
    partitionable = bool(_jc.threefry_partitionable.value)
    kh = np.uint32(np.uint64(seed) >> np.uint64(32))
    kl = np.uint32(seed & 0xFFFFFFFF)
    n = int(np.prod(shape))
    if partitionable:
        b1, b2 = _threefry2x32_raw(kh, kl, np.zeros(2, np.uint32),
                                   np.arange(2, dtype=np.uint32))
        sk1, sk2 = (b1[0], b2[0]), (b1[1], b2[1])

        def bits(k):
            c1 = np.zeros(n, np.uint32)
            c2 = np.arange(n, dtype=np.uint32)
            o1, o2 = _threefry2x32_raw(k[0], k[1], c1, c2)
            return (o1 ^ o2).astype(np.uint32)
    else:
        ks = _threefry2x32_flat(kh, kl, np.arange(4, dtype=np.uint32)).reshape(2, 2)
        sk1, sk2 = tuple(ks[0]), tuple(ks[1])

        def bits(k):
            return _threefry2x32_flat(k[0], k[1], np.arange(n, dtype=np.uint32))

    higher, lower = bits(sk1), bits(sk2)
    span = np.uint32(255)
    with np.errstate(over="ignore"):
        mult = np.uint32(np.uint32(2 ** 16) % span)
        mult = np.uint32((mult * mult) % span)
        off = ((higher % span) * mult + (lower % span)).astype(np.uint32)
        off = (off % span).astype(np.uint32)
    return off.astype(np.int32).reshape(shape)


@functools.cache
def _neg_indices_const():
    neg = _np_randint_0_255((N_B, L, NUM_NEG))
    pos = np.arange(L, dtype=np.int32)[None, :, None]
    neg = (neg + (neg >= pos)).astype(np.int32)
    negp = np.zeros((ROWS, JPAD), np.int32)
    negp[:, 0] = L
    negp[:, 1:NUM_NEG + 1] = neg.reshape(ROWS, NUM_NEG)
    return negp.reshape(-1)


def kernel(x, W_conv, b_conv, mask_token, start_token, W_qkv, W_o):
    xr = x.reshape(N_B, C_IN, L, KERNEL).transpose(0, 2, 1, 3) \
         .reshape(N_B, L, C_IN * KERNEL)
    wc = W_conv.reshape(C_FEAT, C_IN * KERNEL).T
    mcol = jnp.asarray(_MASK_NP[0], jnp.float32)[:, None]

    zT, c_out, M = _tc(xr, wc, b_conv[None, :], mask_token[None, :],
                       start_token[None, :], mcol, W_qkv, W_o)

    gat = _sc_gather_kernel()(M.reshape(-1), _neg_indices_const())
    logits = gat.reshape(ROWS, JPAD)[:, :NUM_NEG + 1]

    return logits, zT, jnp.asarray(_MASK_NP), c_out


# speedup vs baseline: 1.0858x; 1.0858x over previous
import functools

import jax
import jax.numpy as jnp
import numpy as np
from jax import lax
from jax.experimental import pallas as pl
from jax.experimental.pallas import tpu as pltpu
from jax.experimental.pallas import tpu_sc as plsc

N_B, C_IN, T_LEN = 8, 20, 16384
C_FEAT, KERNEL, STRIDE = 256, 64, 64
L = T_LEN // STRIDE
N_HEADS, DH = 8, 32
TEMP, NUM_NEG, EPS = 0.5, 100, 1e-8
P_TOK = 384
LW = 264
JPAD = 112
ROWS = N_B * L
N_TILES = 32
RPT = ROWS // N_TILES


def _eval_mask_np():
    mask = np.zeros((N_B, L), dtype=bool)
    half = max(1, int(L * 0.1 * 0.5))
    seeds = (L // half) * np.arange(half, dtype=np.int64)
    inds = set()
    for s in seeds:
        for i in range(int(s), int(s) + 6):
            if i < L:
                inds.add(i)
    mask[:, sorted(inds)] = True
    return mask


_MASK_NP = _eval_mask_np()


def _tc_body(x_ref, wc_ref, bc_ref, mt_ref, st_ref, mcol_ref, wqkv_ref,
             wo_ref, zT_ref, coT_ref, m_ref):
    f32 = jnp.float32
    xb = x_ref[0]
    wc = wc_ref[...]
    acc = None
    for g in range(C_IN // 4):
        xg = jnp.concatenate(
            [xb[(4 * g + i) * L:(4 * g + i + 1) * L, :] for i in range(4)],
            axis=1)
        wg = wc[:, 256 * g:256 * (g + 1)]
        t = lax.dot_general(xg, wg, (((1,), (1,)), ((), ())),
                            preferred_element_type=f32)
        acc = t if acc is None else acc + t
    zt = jax.nn.gelu(acc + bc_ref[...])
    zT_ref[0] = zt.T

    mcol = mcol_ref[...]
    tokmid = zt * (1.0 - mcol) + mt_ref[...] * mcol
    tok = jnp.concatenate(
        [st_ref[...], tokmid, jnp.zeros((P_TOK - 1 - L, C_FEAT), f32)],
        axis=0)
    qkv = jnp.dot(tok, wqkv_ref[...], preferred_element_type=f32)
    q = qkv[:, 0:C_FEAT]
    k = qkv[:, C_FEAT:2 * C_FEAT]
    v = qkv[:, 2 * C_FEAT:3 * C_FEAT]

    scale = f32(1.0 / np.sqrt(DH))
    o_parts = []
    for h in range(N_HEADS):
        sl = slice(h * DH, (h + 1) * DH)
        qh, kh, vh = q[:, sl], k[:, sl], v[:, sl]
        sc = lax.dot_general(qh, kh, (((1,), (1,)), ((), ())),
                             preferred_element_type=f32) * scale
        e = jnp.exp(sc)
        s = jnp.sum(e, axis=1, keepdims=True) - f32(P_TOK - 1 - L)
        ev = jnp.dot(e, vh, preferred_element_type=f32)
        o_parts.append(ev / s)
    o = jnp.concatenate(o_parts, axis=1)

    wo = wo_ref[...]
    co = jnp.dot(o, wo, preferred_element_type=f32)
    coT = lax.dot_general(wo, o, (((0,), (1,)), ((), ())),
                          preferred_element_type=f32)
    coT_ref[0] = coT[:, :L + 1]

    cq = co[1:L + 1, :]
    S = lax.dot_general(cq, zt, (((1,), (1,)), ((), ())),
                        preferred_element_type=f32)
    ones = jnp.ones((1, L), f32)
    zn2 = lax.dot_general(ones, zt * zt, (((1,), (1,)), ((), ())),
                          preferred_element_type=f32)
    cq2 = jnp.sum(cq * cq, axis=1, keepdims=True)
    den = jnp.maximum(jnp.sqrt(cq2) * jnp.sqrt(zn2), f32(EPS))
    Sp = S / den * f32(1.0 / TEMP)
    T = cq2 + zn2 - 2.0 * S
    eqm = T < (cq2 + zn2) * f32(1e-6)
    Mneg = jnp.where(eqm, f32(-jnp.inf), Sp)
    eye = (lax.broadcasted_iota(jnp.int32, (L, L), 0)
           == lax.broadcasted_iota(jnp.int32, (L, L), 1))
    diag = jnp.sum(jnp.where(eye, Sp, f32(0.0)), axis=1, keepdims=True)
    m_ref[0] = jnp.concatenate(
        [Mneg, diag, jnp.zeros((L, LW - L - 1), f32)], axis=1)


def _tc(xr, wc, bc, mt, st, mcol, wqkv, wo):
    return pl.pallas_call(
        _tc_body,
        grid=(N_B,),
        in_specs=[
            pl.BlockSpec((1, C_IN * L, KERNEL), lambda i: (i, 0, 0)),
            pl.BlockSpec((C_FEAT, C_IN * KERNEL), lambda i: (0, 0)),
            pl.BlockSpec((1, C_FEAT), lambda i: (0, 0)),
            pl.BlockSpec((1, C_FEAT), lambda i: (0, 0)),
            pl.BlockSpec((1, C_FEAT), lambda i: (0, 0)),
            pl.BlockSpec((L, 1), lambda i: (0, 0)),
            pl.BlockSpec((C_FEAT, 3 * C_FEAT), lambda i: (0, 0)),
            pl.BlockSpec((C_FEAT, C_FEAT), lambda i: (0, 0)),
        ],
        out_specs=[
            pl.BlockSpec((1, C_FEAT, L), lambda i: (i, 0, 0)),
            pl.BlockSpec((1, C_FEAT, L + 1), lambda i: (i, 0, 0)),
            pl.BlockSpec((1, L, LW), lambda i: (i, 0, 0)),
        ],
        out_shape=[
            jax.ShapeDtypeStruct((N_B, C_FEAT, L), jnp.float32),
            jax.ShapeDtypeStruct((N_B, C_FEAT, L + 1), jnp.float32),
            jax.ShapeDtypeStruct((N_B, L, LW), jnp.float32),
        ],
        compiler_params=pltpu.CompilerParams(
            dimension_semantics=("arbitrary",)),
    )(xr, wc, bc, mt, st, mcol, wqkv, wo)


_WORDS_M = RPT * LW
_WORDS_J = RPT * JPAD


@functools.cache
def _sc_gather_kernel():
    mesh = plsc.VectorSubcoreMesh(core_axis_name="c", subcore_axis_name="s")

    @functools.partial(
        pl.kernel,
        mesh=mesh,
        out_type=jax.ShapeDtypeStruct((ROWS * JPAD,), jnp.float32),
        scratch_types=[
            pltpu.VMEM((_WORDS_M,), jnp.float32),
            pltpu.VMEM((_WORDS_J,), jnp.int32),
            pltpu.VMEM((_WORDS_J,), jnp.float32),
        ],
        compiler_params=pltpu.CompilerParams(needs_layout_passes=False),
    )
    def sc_gather(m_hbm, neg_hbm, out_hbm, m_v, neg_v, out_v):
        nc = 2
        wid = lax.axis_index("s") * nc + lax.axis_index("c")
        pltpu.sync_copy(m_hbm.at[pl.ds(wid * _WORDS_M, _WORDS_M)], m_v)
        pltpu.sync_copy(neg_hbm.at[pl.ds(wid * _WORDS_J, _WORDS_J)], neg_v)

        def body(r, carry):
            rb = r * JPAD
            off = r * LW
            for j in range(JPAD // 16):
                idx = neg_v[pl.ds(rb + 16 * j, 16)] + off
                out_v[pl.ds(rb + 16 * j, 16)] = plsc.load_gather(m_v, [idx])
            return carry

        lax.fori_loop(0, RPT, body, 0)
        pltpu.sync_copy(out_v, out_hbm.at[pl.ds(wid * _WORDS_J, _WORDS_J)])

    return sc_gather


def _rotl32(x, r):
    return ((x << np.uint32(r)) | (x >> np.uint32(32 - r))).astype(np.uint32)


def _threefry2x32_raw(k1, k2, x1, x2):
    x = [x1.astype(np.uint32).copy(), x2.astype(np.uint32).copy()]
    k1, k2 = np.uint32(k1), np.uint32(k2)
    ks = [k1, k2, np.uint32(k1 ^ k2 ^ np.uint32(0x1BD11BDA))]
    rotations = [(13, 15, 26, 6), (17, 29, 16, 24)]
    with np.errstate(over="ignore"):
        x[0] = (x[0] + ks[0]).astype(np.uint32)
        x[1] = (x[1] + ks[1]).astype(np.uint32)
        for i in range(5):
            for r in rotations[i % 2]:
                x[0] = (x[0] + x[1]).astype(np.uint32)
                x[1] = _rotl32(x[1], r)
                x[1] = (x[1] ^ x[0]).astype(np.uint32)
            x[0] = (x[0] + ks[(i + 1) % 3]).astype(np.uint32)
            x[1] = (x[1] + ks[(i + 2) % 3] + np.uint32(i + 1)).astype(np.uint32)
    return x[0], x[1]


def _threefry2x32_flat(k1, k2, count):
    odd = count.size % 2
    flat = count.ravel().astype(np.uint32)
    if odd:
        flat = np.concatenate([flat, np.zeros(1, np.uint32)])
    h = flat.size // 2
    b1, b2 = _threefry2x32_raw(k1, k2, flat[:h], flat[h:])
    out = np.concatenate([b1, b2])
    if odd:
        out = out[:-1]
    return out


def _np_randint_0_255(shape, seed=1):
    import jax._src.config as _jc
    partitionable = bool(_jc.threefry_partitionable.value)
    kh = np.uint32(np.uint64(seed) >> np.uint64(32))
    kl = np.uint32(seed & 0xFFFFFFFF)
    n = int(np.prod(shape))
    if partitionable:
        b1, b2 = _threefry2x32_raw(kh, kl, np.zeros(2, np.uint32),
                                   np.arange(2, dtype=np.uint32))
        sk1, sk2 = (b1[0], b2[0]), (b1[1], b2[1])

        def bits(k):
            c1 = np.zeros(n, np.uint32)
            c2 = np.arange(n, dtype=np.uint32)
            o1, o2 = _threefry2x32_raw(k[0], k[1], c1, c2)
            return (o1 ^ o2).astype(np.uint32)
    else:
        ks = _threefry2x32_flat(kh, kl, np.arange(4, dtype=np.uint32)).reshape(2, 2)
        sk1, sk2 = tuple(ks[0]), tuple(ks[1])

        def bits(k):
            return _threefry2x32_flat(k[0], k[1], np.arange(n, dtype=np.uint32))

    higher, lower = bits(sk1), bits(sk2)
    span = np.uint32(255)
    with np.errstate(over="ignore"):
        mult = np.uint32(np.uint32(2 ** 16) % span)
        mult = np.uint32((mult * mult) % span)
        off = ((higher % span) * mult + (lower % span)).astype(np.uint32)
        off = (off % span).astype(np.uint32)
    return off.astype(np.int32).reshape(shape)


@functools.cache
def _neg_indices_const():
    neg = _np_randint_0_255((N_B, L, NUM_NEG))
    pos = np.arange(L, dtype=np.int32)[None, :, None]
    neg = (neg + (neg >= pos)).astype(np.int32)
    negp = np.zeros((ROWS, JPAD), np.int32)
    negp[:, 0] = L
    negp[:, 1:NUM_NEG + 1] = neg.reshape(ROWS, NUM_NEG)
    return negp.reshape(-1)


def kernel(x, W_conv, b_conv, mask_token, start_token, W_qkv, W_o):
    xr = x.reshape(N_B, C_IN * L, KERNEL)
    wc = W_conv.reshape(C_FEAT, C_IN * KERNEL)
    mcol = jnp.asarray(_MASK_NP[0], jnp.float32)[:, None]

    zT, c_out, M = _tc(xr, wc, b_conv[None, :], mask_token[None, :],
                       start_token[None, :], mcol, W_qkv, W_o)

    gat = _sc_gather_kernel()(M.reshape(-1), _neg_indices_const())
    logits = gat.reshape(ROWS, JPAD)[:, :NUM_NEG + 1]

    return logits, zT, jnp.asarray(_MASK_NP), c_out


# speedup vs baseline: 1.0874x; 1.0015x over previous
import functools

import jax
import jax.numpy as jnp
import numpy as np
from jax import lax
from jax.experimental import pallas as pl
from jax.experimental.pallas import tpu as pltpu
from jax.experimental.pallas import tpu_sc as plsc

N_B, C_IN, T_LEN = 8, 20, 16384
C_FEAT, KERNEL, STRIDE = 256, 64, 64
L = T_LEN // STRIDE
N_HEADS, DH = 8, 32
TEMP, NUM_NEG, EPS = 0.5, 100, 1e-8
P_TOK = 384
LW = 264
JPAD = 112
ROWS = N_B * L
N_TILES = 32
RPT = ROWS // N_TILES


def _eval_mask_np():
    mask = np.zeros((N_B, L), dtype=bool)
    half = max(1, int(L * 0.1 * 0.5))
    seeds = (L // half) * np.arange(half, dtype=np.int64)
    inds = set()
    for s in seeds:
        for i in range(int(s), int(s) + 6):
            if i < L:
                inds.add(i)
    mask[:, sorted(inds)] = True
    return mask


_MASK_NP = _eval_mask_np()


def _tc_body(x_ref, wc_ref, bc_ref, mt_ref, st_ref, mcol_ref, wqkv_ref,
             wo_ref, zT_ref, coT_ref, m_ref):
    f32 = jnp.float32
    xb = x_ref[0]
    wc = wc_ref[...]
    acc = None
    for g in range(C_IN // 4):
        xg = jnp.concatenate(
            [xb[(4 * g + i) * L:(4 * g + i + 1) * L, :] for i in range(4)],
            axis=1)
        wg = wc[:, 256 * g:256 * (g + 1)]
        t = lax.dot_general(xg, wg, (((1,), (1,)), ((), ())),
                            preferred_element_type=f32)
        acc = t if acc is None else acc + t
    zt = jax.nn.gelu(acc + bc_ref[...])
    zT_ref[0] = zt.T

    mcol = mcol_ref[...]
    tokmid = zt * (1.0 - mcol) + mt_ref[...] * mcol
    tok = jnp.concatenate(
        [st_ref[...], tokmid, jnp.zeros((P_TOK - 1 - L, C_FEAT), f32)],
        axis=0)
    qkv = jnp.dot(tok, wqkv_ref[...], preferred_element_type=f32)
    q = qkv[:, 0:C_FEAT]
    k = qkv[:, C_FEAT:2 * C_FEAT]
    v = qkv[:, 2 * C_FEAT:3 * C_FEAT]

    scale = f32(1.0 / np.sqrt(DH))
    o_parts = []
    for h in range(N_HEADS):
        sl = slice(h * DH, (h + 1) * DH)
        qh, kh, vh = q[:, sl], k[:, sl], v[:, sl]
        sc = lax.dot_general(qh, kh, (((1,), (1,)), ((), ())),
                             preferred_element_type=f32) * scale
        e = jnp.exp(sc)
        s = jnp.sum(e, axis=1, keepdims=True) - f32(P_TOK - 1 - L)
        ev = jnp.dot(e, vh, preferred_element_type=f32)
        o_parts.append(ev / s)
    o = jnp.concatenate(o_parts, axis=1)

    wo = wo_ref[...]
    co = jnp.dot(o, wo, preferred_element_type=f32)
    coT = lax.dot_general(wo, o, (((0,), (1,)), ((), ())),
                          preferred_element_type=f32)
    coT_ref[0] = coT[:, :L + 1]

    cq = co[1:L + 1, :]
    S = lax.dot_general(cq, zt, (((1,), (1,)), ((), ())),
                        preferred_element_type=f32)
    ones = jnp.ones((1, L), f32)
    zn2 = lax.dot_general(ones, zt * zt, (((1,), (1,)), ((), ())),
                          preferred_element_type=f32)
    cq2 = jnp.sum(cq * cq, axis=1, keepdims=True)
    den = jnp.maximum(jnp.sqrt(cq2) * jnp.sqrt(zn2), f32(EPS))
    Sp = S / den * f32(1.0 / TEMP)
    T = cq2 + zn2 - 2.0 * S
    eqm = T < (cq2 + zn2) * f32(1e-6)
    Mneg = jnp.where(eqm, f32(-jnp.inf), Sp)
    eye = (lax.broadcasted_iota(jnp.int32, (L, L), 0)
           == lax.broadcasted_iota(jnp.int32, (L, L), 1))
    diag = jnp.sum(jnp.where(eye, Sp, f32(0.0)), axis=1, keepdims=True)
    m_ref[0] = jnp.concatenate(
        [Mneg, diag, jnp.zeros((L, LW - L - 1), f32)], axis=1)


def _tc(xr, wc, bc, mt, st, mcol, wqkv, wo):
    return pl.pallas_call(
        _tc_body,
        grid=(N_B,),
        in_specs=[
            pl.BlockSpec((1, C_IN * L, KERNEL), lambda i: (i, 0, 0)),
            pl.BlockSpec((C_FEAT, C_IN * KERNEL), lambda i: (0, 0)),
            pl.BlockSpec((1, C_FEAT), lambda i: (0, 0)),
            pl.BlockSpec((1, C_FEAT), lambda i: (0, 0)),
            pl.BlockSpec((1, C_FEAT), lambda i: (0, 0)),
            pl.BlockSpec((L, 1), lambda i: (0, 0)),
            pl.BlockSpec((C_FEAT, 3 * C_FEAT), lambda i: (0, 0)),
            pl.BlockSpec((C_FEAT, C_FEAT), lambda i: (0, 0)),
        ],
        out_specs=[
            pl.BlockSpec((1, C_FEAT, L), lambda i: (i, 0, 0)),
            pl.BlockSpec((1, C_FEAT, L + 1), lambda i: (i, 0, 0)),
            pl.BlockSpec((1, L, LW), lambda i: (i, 0, 0)),
        ],
        out_shape=[
            jax.ShapeDtypeStruct((N_B, C_FEAT, L), jnp.float32),
            jax.ShapeDtypeStruct((N_B, C_FEAT, L + 1), jnp.float32),
            jax.ShapeDtypeStruct((N_B, L, LW), jnp.float32),
        ],
        compiler_params=pltpu.CompilerParams(
            dimension_semantics=("arbitrary",)),
    )(xr, wc, bc, mt, st, mcol, wqkv, wo)


_WORDS_M = RPT * LW
_WORDS_J = RPT * JPAD


@functools.cache
def _sc_gather_kernel():
    mesh = plsc.VectorSubcoreMesh(core_axis_name="c", subcore_axis_name="s")

    @functools.partial(
        pl.kernel,
        mesh=mesh,
        out_type=jax.ShapeDtypeStruct((ROWS * JPAD,), jnp.float32),
        scratch_types=[
            pltpu.VMEM((_WORDS_M,), jnp.float32),
            pltpu.VMEM((_WORDS_J,), jnp.int32),
            pltpu.VMEM((_WORDS_J,), jnp.float32),
        ],
        compiler_params=pltpu.CompilerParams(needs_layout_passes=False),
    )
    def sc_gather(m_hbm, neg_hbm, out_hbm, m_v, neg_v, out_v):
        nc = 2
        wid = lax.axis_index("s") * nc + lax.axis_index("c")
        pltpu.sync_copy(m_hbm.at[pl.ds(wid * _WORDS_M, _WORDS_M)], m_v)
        pltpu.sync_copy(neg_hbm.at[pl.ds(wid * _WORDS_J, _WORDS_J)], neg_v)

        def body(r, carry):
            rb = r * JPAD
            off = r * LW
            for j in range(JPAD // 16):
                idx = neg_v[pl.ds(rb + 16 * j, 16)] + off
                out_v[pl.ds(rb + 16 * j, 16)] = plsc.load_gather(m_v, [idx])
            return carry

        lax.fori_loop(0, RPT, body, 0)
        pltpu.sync_copy(out_v, out_hbm.at[pl.ds(wid * _WORDS_J, _WORDS_J)])

    return sc_gather


def _rotl32(x, r):
    return ((x << np.uint32(r)) | (x >> np.uint32(32 - r))).astype(np.uint32)


def _threefry2x32_raw(k1, k2, x1, x2):
    x = [x1.astype(np.uint32).copy(), x2.astype(np.uint32).copy()]
    k1, k2 = np.uint32(k1), np.uint32(k2)
    ks = [k1, k2, np.uint32(k1 ^ k2 ^ np.uint32(0x1BD11BDA))]
    rotations = [(13, 15, 26, 6), (17, 29, 16, 24)]
    with np.errstate(over="ignore"):
        x[0] = (x[0] + ks[0]).astype(np.uint32)
        x[1] = (x[1] + ks[1]).astype(np.uint32)
        for i in range(5):
            for r in rotations[i % 2]:
                x[0] = (x[0] + x[1]).astype(np.uint32)
                x[1] = _rotl32(x[1], r)
                x[1] = (x[1] ^ x[0]).astype(np.uint32)
            x[0] = (x[0] + ks[(i + 1) % 3]).astype(np.uint32)
            x[1] = (x[1] + ks[(i + 2) % 3] + np.uint32(i + 1)).astype(np.uint32)
    return x[0], x[1]


def _threefry2x32_flat(k1, k2, count):
    odd = count.size % 2
    flat = count.ravel().astype(np.uint32)
    if odd:
        flat = np.concatenate([flat, np.zeros(1, np.uint32)])
    h = flat.size // 2
    b1, b2 = _threefry2x32_raw(k1, k2, flat[:h], flat[h:])
    out = np.concatenate([b1, b2])
    if odd:
        out = out[:-1]
    return out


def _np_randint_0_255(shape, seed=1):
    try:
        import jax._src.config as _jc
        partitionable = bool(_jc.threefry_partitionable.value)
    except Exception:
        partitionable = True
    kh = np.uint32(np.uint64(seed) >> np.uint64(32))
    kl = np.uint32(seed & 0xFFFFFFFF)
    n = int(np.prod(shape))
    if partitionable:
        b1, b2 = _threefry2x32_raw(kh, kl, np.zeros(2, np.uint32),
                                   np.arange(2, dtype=np.uint32))
        sk1, sk2 = (b1[0], b2[0]), (b1[1], b2[1])

        def bits(k):
            c1 = np.zeros(n, np.uint32)
            c2 = np.arange(n, dtype=np.uint32)
            o1, o2 = _threefry2x32_raw(k[0], k[1], c1, c2)
            return (o1 ^ o2).astype(np.uint32)
    else:
        ks = _threefry2x32_flat(kh, kl, np.arange(4, dtype=np.uint32)).reshape(2, 2)
        sk1, sk2 = tuple(ks[0]), tuple(ks[1])

        def bits(k):
            return _threefry2x32_flat(k[0], k[1], np.arange(n, dtype=np.uint32))

    higher, lower = bits(sk1), bits(sk2)
    span = np.uint32(255)
    with np.errstate(over="ignore"):
        mult = np.uint32(np.uint32(2 ** 16) % span)
        mult = np.uint32((mult * mult) % span)
        off = ((higher % span) * mult + (lower % span)).astype(np.uint32)
        off = (off % span).astype(np.uint32)
    return off.astype(np.int32).reshape(shape)


@functools.cache
def _neg_indices_const():
    neg = _np_randint_0_255((N_B, L, NUM_NEG))
    pos = np.arange(L, dtype=np.int32)[None, :, None]
    neg = (neg + (neg >= pos)).astype(np.int32)
    negp = np.zeros((ROWS, JPAD), np.int32)
    negp[:, 0] = L
    negp[:, 1:NUM_NEG + 1] = neg.reshape(ROWS, NUM_NEG)
    return negp.reshape(-1)


def kernel(x, W_conv, b_conv, mask_token, start_token, W_qkv, W_o):
    xr = x.reshape(N_B, C_IN * L, KERNEL)
    wc = W_conv.reshape(C_FEAT, C_IN * KERNEL)
    mcol = jnp.asarray(_MASK_NP[0], jnp.float32)[:, None]

    zT, c_out, M = _tc(xr, wc, b_conv[None, :], mask_token[None, :],
                       start_token[None, :], mcol, W_qkv, W_o)

    gat = _sc_gather_kernel()(M.reshape(-1), _neg_indices_const())
    logits = gat.reshape(ROWS, JPAD)[:, :NUM_NEG + 1]

    return logits, zT, jnp.asarray(_MASK_NP), c_out
